# baseline (speedup 1.0000x reference)
import functools

import jax
import jax.numpy as jnp
from jax import lax
from jax.experimental import pallas as pl
from jax.experimental.pallas import tpu as pltpu
from jax.experimental.pallas import tpu_sc as plsc

_NUM_CORES = 2
_NUM_SUBCORES = 16
_NUM_WORKERS = _NUM_CORES * _NUM_SUBCORES
_L = 16
_CHUNK = 16
_NBUF = 4


def kernel(data, partitions):
    n_rows, n_cols = data.shape
    partitions = partitions.astype(jnp.int32)
    rows_per_w = n_rows // _NUM_WORKERS
    n_vecs_total = n_rows // _L
    n_vecs_w = rows_per_w // _L
    n_chunks = rows_per_w // _CHUNK
    n_groups = n_chunks // _NBUF
    mesh = plsc.VectorSubcoreMesh(
        core_axis_name="c", subcore_axis_name="s",
        num_cores=_NUM_CORES, num_subcores=_NUM_SUBCORES)

    @functools.partial(
        pl.kernel,
        mesh=mesh,
        compiler_params=pltpu.CompilerParams(needs_layout_passes=False),
        out_type=jax.ShapeDtypeStruct((n_rows, n_cols), data.dtype),
        scratch_types=[
            pltpu.VMEM((n_rows,), jnp.int32),
            pltpu.VMEM((n_chunks, _CHUNK), jnp.int32),
            pltpu.VMEM((_NBUF, _CHUNK, n_cols), jnp.float32),
            pltpu.SemaphoreType.DMA,
            pltpu.SemaphoreType.DMA,
            pltpu.SemaphoreType.DMA,
            pltpu.SemaphoreType.DMA,
            pltpu.SemaphoreType.DMA,
            pltpu.SemaphoreType.DMA,
            pltpu.SemaphoreType.DMA,
            pltpu.SemaphoreType.DMA,
        ],
    )
    def run(data_hbm, part_hbm, out_hbm, pv, idx, buf, *sems):
        sin = sems[:_NBUF]
        sout = sems[_NBUF:]
        wid = lax.axis_index("s") * _NUM_CORES + lax.axis_index("c")
        base = wid * rows_per_w
        base_vec = wid * n_vecs_w

        pltpu.sync_copy(part_hbm, pv)

        def count_body(g, acc):
            return acc + pv[pl.ds(g * _L, _L)]

        pre_v = lax.fori_loop(0, base_vec, count_body,
                              jnp.zeros((_L,), jnp.int32))
        tot_v = lax.fori_loop(base_vec, n_vecs_total, count_body, pre_v)
        ones_before = jnp.sum(pre_v)
        ones_total = jnp.sum(tot_v)
        zeros_total = n_rows - ones_total

        iota = lax.iota(jnp.int32, _L)

        def dest_body(k, ones_run):
            v = pv[pl.ds((base_vec + k) * _L, _L)]
            incl = plsc.cumsum(v)
            ones_excl = ones_run + incl - v
            row = base + k * _L + iota
            dest = jnp.where(v == 0, row - ones_excl, zeros_total + ones_excl)
            idx[k] = dest
            return ones_run + jnp.max(incl)

        lax.fori_loop(0, n_vecs_w, dest_body, ones_before)

        def grp(g, carry):
            for b in range(_NBUF):
                k = g * _NBUF + b
                kp = (g - 1) * _NBUF + b

                @pl.when(g > 0)
                def _():
                    pltpu.make_async_copy(
                        buf.at[b], out_hbm.at[idx.at[kp]], sout[b]).wait()

                pltpu.async_copy(data_hbm.at[idx.at[k]], buf.at[b], sin[b])
            for b in range(_NBUF):
                k = g * _NBUF + b
                pltpu.make_async_copy(
                    data_hbm.at[idx.at[k]], buf.at[b], sin[b]).wait()
                pltpu.async_copy(buf.at[b], out_hbm.at[idx.at[k]], sout[b])
            return carry

        lax.fori_loop(0, n_groups, grp, 0)
        for b in range(_NBUF):
            k = (n_groups - 1) * _NBUF + b
            pltpu.make_async_copy(
                buf.at[b], out_hbm.at[idx.at[k]], sout[b]).wait()

    return run(data, partitions)


# speedup vs baseline: 1.0206x; 1.0206x over previous
import functools

import jax
import jax.numpy as jnp
from jax import lax
from jax.experimental import pallas as pl
from jax.experimental.pallas import tpu as pltpu
from jax.experimental.pallas import tpu_sc as plsc

_NUM_CORES = 2
_NUM_SUBCORES = 16
_NUM_WORKERS = _NUM_CORES * _NUM_SUBCORES
_L = 16
_CHUNK = 32
_NBUF = 2
_UNROLL = 8


def kernel(data, partitions):
    n_rows, n_cols = data.shape
    partitions = partitions.astype(jnp.int32)
    rows_per_w = n_rows // _NUM_WORKERS
    n_vecs_total = n_rows // _L
    n_vecs_w = rows_per_w // _L
    n_chunks = rows_per_w // _CHUNK
    n_groups = n_chunks // _NBUF
    vecs_per_chunk = _CHUNK // _L
    mesh = plsc.VectorSubcoreMesh(
        core_axis_name="c", subcore_axis_name="s",
        num_cores=_NUM_CORES, num_subcores=_NUM_SUBCORES)

    @functools.partial(
        pl.kernel,
        mesh=mesh,
        compiler_params=pltpu.CompilerParams(needs_layout_passes=False),
        out_type=jax.ShapeDtypeStruct((n_rows, n_cols), data.dtype),
        scratch_types=[
            pltpu.VMEM((n_rows,), jnp.int32),
            pltpu.VMEM((n_chunks, _CHUNK), jnp.int32),
            pltpu.VMEM((_NBUF, _CHUNK, n_cols), jnp.float32),
            pltpu.SemaphoreType.DMA,
            pltpu.SemaphoreType.DMA,
            pltpu.SemaphoreType.DMA,
            pltpu.SemaphoreType.DMA,
        ],
    )
    def run(data_hbm, part_hbm, out_hbm, pv, idx, buf, *sems):
        sin = sems[:_NBUF]
        sout = sems[_NBUF:]
        wid = lax.axis_index("s") * _NUM_CORES + lax.axis_index("c")
        base = wid * rows_per_w
        base_vec = wid * n_vecs_w

        pltpu.sync_copy(part_hbm, pv)

        zero = jnp.zeros((_L,), jnp.int32)

        def count_body(g, accs):
            new = []
            for j, a in enumerate(accs):
                off = (g * _UNROLL + 2 * j) * _L
                a = a + pv[pl.ds(off, _L)] + pv[pl.ds(off + _L, _L)]
                new.append(a)
            return tuple(new)

        accs0 = (zero,) * (_UNROLL // 2)
        pre_accs = lax.fori_loop(0, base_vec // _UNROLL, count_body, accs0)
        tot_accs = lax.fori_loop(base_vec // _UNROLL, n_vecs_total // _UNROLL,
                                 count_body, pre_accs)
        ones_before = jnp.sum(sum(pre_accs, zero))
        ones_total = jnp.sum(sum(tot_accs, zero))
        zeros_total = n_rows - ones_total

        iota = lax.iota(jnp.int32, _L)

        def dest_body(k, ones_run):
            for j in range(vecs_per_chunk):
                kv = k * vecs_per_chunk + j
                v = pv[pl.ds((base_vec + kv) * _L, _L)]
                incl = plsc.cumsum(v)
                ones_excl = ones_run + incl - v
                row = base + kv * _L + iota
                dest = jnp.where(v == 0, row - ones_excl,
                                 zeros_total + ones_excl)
                idx[k, pl.ds(j * _L, _L)] = dest
                ones_run = ones_run + jnp.max(incl)
            return ones_run

        lax.fori_loop(0, n_chunks, dest_body, ones_before)

        def grp(g, carry):
            for b in range(_NBUF):
                k = g * _NBUF + b
                kp = (g - 1) * _NBUF + b

                @pl.when(g > 0)
                def _():
                    pltpu.make_async_copy(
                        buf.at[b], out_hbm.at[idx.at[kp]], sout[b]).wait()

                pltpu.async_copy(data_hbm.at[idx.at[k]], buf.at[b], sin[b])
            for b in range(_NBUF):
                k = g * _NBUF + b
                pltpu.make_async_copy(
                    data_hbm.at[idx.at[k]], buf.at[b], sin[b]).wait()
                pltpu.async_copy(buf.at[b], out_hbm.at[idx.at[k]], sout[b])
            return carry

        lax.fori_loop(0, n_groups, grp, 0)
        for b in range(_NBUF):
            k = (n_groups - 1) * _NBUF + b
            pltpu.make_async_copy(
                buf.at[b], out_hbm.at[idx.at[k]], sout[b]).wait()

    return run(data, partitions)
